# baseline (speedup 1.0000x reference)
import jax
import jax.numpy as jnp
from jax import lax
from jax.experimental import pallas as pl
from jax.experimental.pallas import tpu as pltpu
from jax.experimental.pallas import tpu_sc as plsc

D = 64
LANE = 128
NC, NS = 2, 16
NW = NC * NS
NB = 2
K = 4


def _emb_body(idx_hbm, w_hbm, out_hbm, idx_v, rows_v, gsem):
    tok_rows = idx_hbm.shape[0]
    rpw = tok_rows // NW
    ch = rpw // K

    c = lax.axis_index("c")
    s = lax.axis_index("s")
    wid = s * NC + c
    base = wid * rpw

    def start(g, b):
        r0 = base + g * K
        pltpu.sync_copy(idx_hbm.at[pl.ds(r0, K)], idx_v.at[b])
        for j in range(K):
            pltpu.async_copy(w_hbm.at[idx_v.at[b, j]], rows_v.at[b, j],
                             gsem.at[b])

    def finish(g, b):
        r0 = base + g * K
        pltpu.make_async_copy(out_hbm.at[pl.ds(r0, K)], rows_v.at[b],
                              gsem.at[b]).wait()
        pltpu.sync_copy(rows_v.at[b], out_hbm.at[pl.ds(r0, K)])

    start(0, 0)
    start(1, 1)

    def loop_body(i, carry):
        g = 2 * i
        finish(g, 0)
        start(g + 2, 0)
        finish(g + 1, 1)
        start(g + 3, 1)
        return carry

    lax.fori_loop(0, (ch - 2) // 2, loop_body, 0)
    finish(ch - 2, 0)
    finish(ch - 1, 1)


def kernel(tokens, weight):
    b0, b1 = tokens.shape
    tok_rows = (b0 * b1) // LANE
    idx = tokens.reshape(tok_rows, LANE)
    mesh = plsc.VectorSubcoreMesh(core_axis_name="c", subcore_axis_name="s",
                                  num_cores=NC, num_subcores=NS)
    out = pl.kernel(
        _emb_body,
        out_type=jax.ShapeDtypeStruct((tok_rows, LANE, D), jnp.float32),
        mesh=mesh,
        scratch_types=[
            pltpu.VMEM((NB, K, LANE), jnp.int32),
            pltpu.VMEM((NB, K, LANE, D), jnp.float32),
            pltpu.SemaphoreType.DMA((NB,)),
        ],
        compiler_params=pltpu.CompilerParams(use_tc_tiling_on_sc=False),
    )(idx, weight)
    return out.reshape(b0, b1, D)


# speedup vs baseline: 1.2279x; 1.2279x over previous
import jax
import jax.numpy as jnp
from jax import lax
from jax.experimental import pallas as pl
from jax.experimental.pallas import tpu as pltpu
from jax.experimental.pallas import tpu_sc as plsc

D = 64
LANE = 128
NC, NS = 2, 16
NW = NC * NS
NB = 2
K = 2


def _emb_body(idx_hbm, w_hbm, out_hbm, idx_v, rows_v, gsem):
    tok_rows = idx_hbm.shape[0]
    rpw = tok_rows // NW
    ch = rpw // K

    c = lax.axis_index("c")
    s = lax.axis_index("s")
    wid = s * NC + c
    base = wid * rpw

    def start(g, b):
        r0 = base + g * K
        pltpu.sync_copy(idx_hbm.at[pl.ds(r0, K)], idx_v.at[b])
        for j in range(K):
            pltpu.async_copy(w_hbm.at[idx_v.at[b, j]], rows_v.at[b, j],
                             gsem.at[b])

    def finish(g, b):
        r0 = base + g * K
        for j in range(K):
            pltpu.make_async_copy(w_hbm.at[pl.ds(0, LANE)], rows_v.at[b, j],
                                  gsem.at[b]).wait()
        pltpu.sync_copy(rows_v.at[b], out_hbm.at[pl.ds(r0, K)])

    start(0, 0)
    start(1, 1)

    def loop_body(i, carry):
        g = 2 * i
        finish(g, 0)
        start(g + 2, 0)
        finish(g + 1, 1)
        start(g + 3, 1)
        return carry

    lax.fori_loop(0, (ch - 2) // 2, loop_body, 0)
    finish(ch - 2, 0)
    finish(ch - 1, 1)


def kernel(tokens, weight):
    b0, b1 = tokens.shape
    vocab = weight.shape[0]
    tok_rows = (b0 * b1) // LANE
    idx = tokens.reshape(tok_rows, LANE)
    w128 = jnp.pad(weight, ((0, 0), (0, LANE - D)))
    mesh = plsc.VectorSubcoreMesh(core_axis_name="c", subcore_axis_name="s",
                                  num_cores=NC, num_subcores=NS)
    out = pl.kernel(
        _emb_body,
        out_type=jax.ShapeDtypeStruct((tok_rows, LANE, LANE), jnp.float32),
        mesh=mesh,
        scratch_types=[
            pltpu.VMEM((NB, K, LANE), jnp.int32),
            pltpu.VMEM((NB, K, LANE, LANE), jnp.float32),
            pltpu.SemaphoreType.DMA((NB,)),
        ],
    )(idx, w128)
    return out[:, :, :D].reshape(b0, b1, D)
